# baseline (speedup 1.0000x reference)
import jax
import jax.numpy as jnp
from jax.experimental import pallas as pl
from jax.experimental.pallas import tpu as pltpu


_TM = 512


def _interp_tile(s0_ref, s1_ref, x_ref, o_ref):
    o_ref[0] = jnp.dot(s0_ref[...], x_ref[...],
                       preferred_element_type=jnp.float32)
    o_ref[1] = jnp.dot(s1_ref[...], x_ref[...],
                       preferred_element_type=jnp.float32)


def kernel(x_coarse, interp_matrix):
    m, k = interp_matrix.shape
    n = x_coarse.shape[1]
    half_steps = m // (2 * _TM)
    out = pl.pallas_call(
        _interp_tile,
        grid=(half_steps,),
        in_specs=[
            pl.BlockSpec((_TM, k), lambda i: (i, 0)),
            pl.BlockSpec((_TM, k), lambda i, hs=half_steps: (i + hs, 0)),
            pl.BlockSpec(memory_space=pltpu.MemorySpace.VMEM),
        ],
        out_specs=pl.BlockSpec((2, _TM, n), lambda i: (0, i, 0)),
        out_shape=jax.ShapeDtypeStruct((2, m // 2, n), jnp.float32),
    )(interp_matrix, interp_matrix, x_coarse)
    return out.reshape(m, n)
